# baseline (speedup 1.0000x reference)
import functools

import jax
import jax.numpy as jnp
from jax import lax
from jax.experimental import pallas as pl
from jax.experimental.pallas import tpu as pltpu
from jax.experimental.pallas import tpu_sc as plsc

N = 10000
E = 320000
D_IN = 128
D_HID = 256
R_PAD = 10112
JUNK = N
NS = 16
ROWS_PER_SUB = R_PAD // NS

DEG_CHUNKS = 80
E_DEG = 32 * DEG_CHUNKS * 128

G1_SC0 = 5
G1_SC1 = 5

_mesh = lambda: plsc.VectorSubcoreMesh(core_axis_name="c", subcore_axis_name="s")


def _sc_degree(cold, zeros128, ones128):
    @functools.partial(
        pl.kernel,
        out_type=jax.ShapeDtypeStruct((2, R_PAD, 128), jnp.float32),
        mesh=_mesh(),
        scratch_types=[
            pltpu.VMEM((DEG_CHUNKS, 128), jnp.int32),
            pltpu.VMEM((128, 128), jnp.float32),
            pltpu.VMEM_SHARED((R_PAD, 128), jnp.float32),
            pltpu.SemaphoreType.DMA,
        ],
    )
    def deg_kernel(col_hbm, z_hbm, ones_hbm, out, cidx, ones_v, acc, dsem):
        cid = lax.axis_index("c")
        sid = lax.axis_index("s")
        w = cid * NS + sid
        pltpu.sync_copy(col_hbm.at[w], cidx)
        pltpu.sync_copy(ones_hbm, ones_v)
        sl = pl.ds(sid * ROWS_PER_SUB, ROWS_PER_SUB)
        pltpu.sync_copy(z_hbm, acc.at[sl])
        plsc.subcore_barrier()

        def wave(t, carry):
            for k in range(8):
                pltpu.async_copy(ones_v, acc.at[cidx.at[t * 8 + k]], dsem,
                                 add=True)
            for k in range(8):
                pltpu.make_async_copy(
                    ones_v, acc.at[cidx.at[t * 8 + k]], dsem).wait()
            return carry

        lax.fori_loop(0, DEG_CHUNKS // 8, wave, 0)
        plsc.subcore_barrier()
        pltpu.sync_copy(acc.at[sl], out.at[cid].at[sl])

    return deg_kernel(cold, zeros128, ones128)


_SPLIT = 4


def _issue_gather(t_hbm, ridx, j, gbuf, sem):
    step = 128 // _SPLIT
    for p in range(_SPLIT):
        pltpu.async_copy(
            t_hbm.at[ridx.at[j].at[pl.ds(p * step, step)]],
            gbuf.at[pl.ds(p * step, step)], sem)


def _wait_gather(t_hbm, ridx, j, gbuf, sem):
    step = 128 // _SPLIT
    for p in range(_SPLIT):
        pltpu.make_async_copy(
            t_hbm.at[ridx.at[j].at[pl.ds(p * step, step)]],
            gbuf.at[pl.ds(p * step, step)], sem).wait()


def _sc_aggregate(row2d, col2d, table, zeros128, g0, g1):
    @functools.partial(
        pl.kernel,
        out_type=jax.ShapeDtypeStruct((2, R_PAD, 128), jnp.float32),
        mesh=_mesh(),
        scratch_types=[
            pltpu.VMEM((16, 128), jnp.int32),
            pltpu.VMEM((16, 128), jnp.int32),
            pltpu.VMEM((128, 128), jnp.float32),
            pltpu.VMEM((128, 128), jnp.float32),
            pltpu.VMEM_SHARED((R_PAD, 128), jnp.float32),
            pltpu.SemaphoreType.DMA,
            pltpu.SemaphoreType.DMA,
            pltpu.SemaphoreType.DMA,
            pltpu.SemaphoreType.DMA,
        ],
    )
    def agg_kernel(row_hbm, col_hbm, t_hbm, z_hbm, out,
                   ridx, cidx, gbuf0, gbuf1, acc, gsem0, gsem1, ssem0, ssem1):
        cid = lax.axis_index("c")
        sid = lax.axis_index("s")
        base = (1 - cid) * (sid * g0 * 16) + cid * (NS * g0 * 16 +
                                                    sid * g1 * 16)
        ngroups = g0 + cid * (g1 - g0)
        sl = pl.ds(sid * ROWS_PER_SUB, ROWS_PER_SUB)
        pltpu.sync_copy(z_hbm, acc.at[sl])
        plsc.subcore_barrier()

        def group(g, carry):
            pltpu.sync_copy(row_hbm.at[pl.ds(base + g * 16, 16)], ridx)
            pltpu.sync_copy(col_hbm.at[pl.ds(base + g * 16, 16)], cidx)
            _issue_gather(t_hbm, ridx, 0, gbuf0, gsem0)
            _issue_gather(t_hbm, ridx, 1, gbuf1, gsem1)

            def steady(t, c2):
                j = 2 * t
                _wait_gather(t_hbm, ridx, j, gbuf0, gsem0)
                pltpu.async_copy(gbuf0, acc.at[cidx.at[j]], ssem0, add=True)
                _wait_gather(t_hbm, ridx, j + 1, gbuf1, gsem1)
                pltpu.async_copy(gbuf1, acc.at[cidx.at[j + 1]], ssem1, add=True)
                pltpu.make_async_copy(gbuf0, acc.at[cidx.at[j]], ssem0).wait()
                _issue_gather(t_hbm, ridx, j + 2, gbuf0, gsem0)
                pltpu.make_async_copy(
                    gbuf1, acc.at[cidx.at[j + 1]], ssem1).wait()
                _issue_gather(t_hbm, ridx, j + 3, gbuf1, gsem1)
                return c2

            lax.fori_loop(0, 7, steady, carry)
            _wait_gather(t_hbm, ridx, 14, gbuf0, gsem0)
            pltpu.sync_copy(gbuf0, acc.at[cidx.at[14]], add=True)
            _wait_gather(t_hbm, ridx, 15, gbuf1, gsem1)
            pltpu.sync_copy(gbuf1, acc.at[cidx.at[15]], add=True)
            return carry

        lax.fori_loop(0, ngroups, group, 0)
        plsc.subcore_barrier()
        pltpu.sync_copy(acc.at[sl], out.at[cid].at[sl])

    return agg_kernel(row2d, col2d, table, zeros128)


def _tc_prescale(d, x):
    def body(d_ref, x_ref, v_ref, dinv_ref):
        deg = d_ref[0, 0:N, 0:1] + d_ref[1, 0:N, 0:1] + 1.0
        dinv = lax.rsqrt(deg)
        dinv_ref[...] = dinv
        v = x_ref[...] * dinv
        v_ref[0:N, :] = v
        v_ref[N:2 * N, :] = v

    return pl.pallas_call(
        body,
        out_shape=(jax.ShapeDtypeStruct((2 * N, 128), jnp.float32),
                   jax.ShapeDtypeStruct((N, 1), jnp.float32)),
    )(d, x)


def _tc_layer1(s1, v1, dinv2d, w1, b1):
    def body(s_ref, v1_ref, dinv_ref, w1_ref, b1_ref, v2_ref):
        dinv = dinv_ref[...]
        ax = (s_ref[0, 0:N, :] + s_ref[1, 0:N, :] + v1_ref[0:N, :]) * dinv
        h = jnp.dot(ax, w1_ref[...], preferred_element_type=jnp.float32)
        h = jnp.maximum(h + b1_ref[...], 0.0) * dinv
        v2_ref[0:N, :] = h[:, 0:128]
        v2_ref[N:2 * N, :] = h[:, 128:256]

    return pl.pallas_call(
        body,
        out_shape=jax.ShapeDtypeStruct((2 * N, 128), jnp.float32),
    )(s1, v1, dinv2d, w1, b1)


def _tc_layer2(s2, v2, dinv2d, w2, b2, wr, br):
    def body(s2_ref, v2_ref, dinv_ref, w2_ref, b2_ref, wr_ref, br_ref, o_ref):
        dinv = dinv_ref[...]
        ah = jnp.concatenate(
            [s2_ref[0, 0:N, :] + v2_ref[0:N, :],
             s2_ref[1, 0:N, :] + v2_ref[N:2 * N, :]], axis=1) * dinv
        z = jnp.dot(ah, w2_ref[...], preferred_element_type=jnp.float32)
        h2 = jnp.maximum(z + b2_ref[...], 0.0)
        logit = jnp.dot(h2, wr_ref[...], preferred_element_type=jnp.float32)
        logit = logit + br_ref[...]
        o_ref[...] = 4.0 / (1.0 + jnp.exp(-logit))

    return pl.pallas_call(
        body,
        out_shape=jax.ShapeDtypeStruct((N, 1), jnp.float32),
    )(s2, v2, dinv2d, w2, b2, wr, br)


def kernel(x, edge_index, W1, b1, W2, b2, Wr, br):
    ei = edge_index.astype(jnp.int32)
    row, col = ei[0], ei[1]

    e1 = 32 * 5 * 16 * 128
    e1_sc0 = 16 * G1_SC0 * 16 * 128
    rowp = jnp.concatenate([row, jnp.zeros((e1 - E,), jnp.int32)])
    colp = jnp.concatenate([col, jnp.full((e1 - E,), JUNK, jnp.int32)])
    row1 = jnp.concatenate(
        [rowp[:e1_sc0], rowp[e1_sc0:] + N]).reshape(-1, 128)
    col1 = colp.reshape(-1, 128)

    row2 = jnp.concatenate([rowp, rowp + N]).reshape(-1, 128)
    col2 = jnp.concatenate([colp, colp]).reshape(-1, 128)

    cold = colp.reshape(32, DEG_CHUNKS, 128)

    zeros128 = jnp.zeros((ROWS_PER_SUB, 128), jnp.float32)
    ones128 = jnp.ones((128, 128), jnp.float32)

    d = _sc_degree(cold, zeros128, ones128)

    v1, dinv2d = _tc_prescale(d, x)
    s1 = _sc_aggregate(row1, col1, v1, zeros128, G1_SC0, G1_SC1)

    v2 = _tc_layer1(s1, v1, dinv2d, W1, b1.reshape(1, D_HID))
    s2 = _sc_aggregate(row2, col2, v2, zeros128, 10, 10)

    return _tc_layer2(s2, v2, dinv2d, W2, b2.reshape(1, D_HID),
                      Wr, br.reshape(1, 1))


# speedup vs baseline: 1.0636x; 1.0636x over previous
import functools

import jax
import jax.numpy as jnp
from jax import lax
from jax.experimental import pallas as pl
from jax.experimental.pallas import tpu as pltpu
from jax.experimental.pallas import tpu_sc as plsc

N = 10000
E = 320000
D_IN = 128
D_HID = 256
R_PAD = 10112
JUNK = N
NS = 16
ROWS_PER_SUB = R_PAD // NS

DEG_CHUNKS = 80
E_DEG = 32 * DEG_CHUNKS * 128

G1_SC0 = 5
G1_SC1 = 5

_mesh = lambda: plsc.VectorSubcoreMesh(core_axis_name="c", subcore_axis_name="s")


def _sc_degree(cold, zeros128, ones128):
    @functools.partial(
        pl.kernel,
        out_type=jax.ShapeDtypeStruct((2, R_PAD, 128), jnp.float32),
        mesh=_mesh(),
        scratch_types=[
            pltpu.VMEM((DEG_CHUNKS, 128), jnp.int32),
            pltpu.VMEM((128, 128), jnp.float32),
            pltpu.VMEM_SHARED((R_PAD, 128), jnp.float32),
            pltpu.SemaphoreType.DMA,
        ],
    )
    def deg_kernel(col_hbm, z_hbm, ones_hbm, out, cidx, ones_v, acc, dsem):
        cid = lax.axis_index("c")
        sid = lax.axis_index("s")
        w = cid * NS + sid
        pltpu.sync_copy(col_hbm.at[w], cidx)
        pltpu.sync_copy(ones_hbm, ones_v)
        sl = pl.ds(sid * ROWS_PER_SUB, ROWS_PER_SUB)
        pltpu.sync_copy(z_hbm, acc.at[sl])
        plsc.subcore_barrier()

        def wave(t, carry):
            for k in range(8):
                pltpu.async_copy(ones_v, acc.at[cidx.at[t * 8 + k]], dsem,
                                 add=True)
            for k in range(8):
                pltpu.make_async_copy(
                    ones_v, acc.at[cidx.at[t * 8 + k]], dsem).wait()
            return carry

        lax.fori_loop(0, DEG_CHUNKS // 8, wave, 0)
        plsc.subcore_barrier()
        pltpu.sync_copy(acc.at[sl], out.at[cid].at[sl])

    return deg_kernel(cold, zeros128, ones128)


_SPLIT = 4


def _issue_gather(t_hbm, ridx, j, gbuf, sem):
    step = 128 // _SPLIT
    for p in range(_SPLIT):
        pltpu.async_copy(
            t_hbm.at[ridx.at[j].at[pl.ds(p * step, step)]],
            gbuf.at[pl.ds(p * step, step)], sem)


def _wait_gather(t_hbm, ridx, j, gbuf, sem):
    step = 128 // _SPLIT
    for p in range(_SPLIT):
        pltpu.make_async_copy(
            t_hbm.at[ridx.at[j].at[pl.ds(p * step, step)]],
            gbuf.at[pl.ds(p * step, step)], sem).wait()


def _sc_aggregate(row2d, col2d, table, zeros128, g0, g1):
    @functools.partial(
        pl.kernel,
        out_type=jax.ShapeDtypeStruct((2, R_PAD, 128), jnp.float32),
        mesh=_mesh(),
        scratch_types=[
            pltpu.VMEM((16, 128), jnp.int32),
            pltpu.VMEM((16, 128), jnp.int32),
            pltpu.VMEM((128, 128), jnp.float32),
            pltpu.VMEM((128, 128), jnp.float32),
            pltpu.VMEM_SHARED((R_PAD, 128), jnp.float32),
            pltpu.SemaphoreType.DMA,
            pltpu.SemaphoreType.DMA,
            pltpu.SemaphoreType.DMA,
            pltpu.SemaphoreType.DMA,
        ],
    )
    def agg_kernel(row_hbm, col_hbm, t_hbm, z_hbm, out,
                   ridx, cidx, gbuf0, gbuf1, acc, gsem0, gsem1, ssem0, ssem1):
        cid = lax.axis_index("c")
        sid = lax.axis_index("s")
        base = (1 - cid) * (sid * g0 * 16) + cid * (NS * g0 * 16 +
                                                    sid * g1 * 16)
        ngroups = g0 + cid * (g1 - g0)
        sl = pl.ds(sid * ROWS_PER_SUB, ROWS_PER_SUB)
        pltpu.sync_copy(z_hbm, acc.at[sl])
        plsc.subcore_barrier()

        def group(g, carry):
            pltpu.sync_copy(row_hbm.at[pl.ds(base + g * 16, 16)], ridx)
            pltpu.sync_copy(col_hbm.at[pl.ds(base + g * 16, 16)], cidx)
            _issue_gather(t_hbm, ridx, 0, gbuf0, gsem0)
            _issue_gather(t_hbm, ridx, 1, gbuf1, gsem1)

            def steady(t, c2):
                j = 2 * t
                _wait_gather(t_hbm, ridx, j, gbuf0, gsem0)
                pltpu.async_copy(gbuf0, acc.at[cidx.at[j]], ssem0, add=True)
                pltpu.make_async_copy(gbuf0, acc.at[cidx.at[j]], ssem0).wait()
                _issue_gather(t_hbm, ridx, j + 2, gbuf0, gsem0)
                _wait_gather(t_hbm, ridx, j + 1, gbuf1, gsem1)
                pltpu.async_copy(gbuf1, acc.at[cidx.at[j + 1]], ssem1, add=True)
                pltpu.make_async_copy(
                    gbuf1, acc.at[cidx.at[j + 1]], ssem1).wait()
                _issue_gather(t_hbm, ridx, j + 3, gbuf1, gsem1)
                return c2

            lax.fori_loop(0, 7, steady, carry)
            _wait_gather(t_hbm, ridx, 14, gbuf0, gsem0)
            pltpu.sync_copy(gbuf0, acc.at[cidx.at[14]], add=True)
            _wait_gather(t_hbm, ridx, 15, gbuf1, gsem1)
            pltpu.sync_copy(gbuf1, acc.at[cidx.at[15]], add=True)
            return carry

        lax.fori_loop(0, ngroups, group, 0)
        plsc.subcore_barrier()
        pltpu.sync_copy(acc.at[sl], out.at[cid].at[sl])

    return agg_kernel(row2d, col2d, table, zeros128)


def _tc_prescale(d, x):
    def body(d_ref, x_ref, v_ref, dinv_ref):
        deg = d_ref[0, 0:N, 0:1] + d_ref[1, 0:N, 0:1] + 1.0
        dinv = lax.rsqrt(deg)
        dinv_ref[...] = dinv
        v = x_ref[...] * dinv
        v_ref[0:N, :] = v
        v_ref[N:2 * N, :] = v

    return pl.pallas_call(
        body,
        out_shape=(jax.ShapeDtypeStruct((2 * N, 128), jnp.float32),
                   jax.ShapeDtypeStruct((N, 1), jnp.float32)),
    )(d, x)


def _tc_layer1(s1, v1, dinv2d, w1, b1):
    def body(s_ref, v1_ref, dinv_ref, w1_ref, b1_ref, v2_ref):
        dinv = dinv_ref[...]
        ax = (s_ref[0, 0:N, :] + s_ref[1, 0:N, :] + v1_ref[0:N, :]) * dinv
        h = jnp.dot(ax, w1_ref[...], preferred_element_type=jnp.float32)
        h = jnp.maximum(h + b1_ref[...], 0.0) * dinv
        v2_ref[0:N, :] = h[:, 0:128]
        v2_ref[N:2 * N, :] = h[:, 128:256]

    return pl.pallas_call(
        body,
        out_shape=jax.ShapeDtypeStruct((2 * N, 128), jnp.float32),
    )(s1, v1, dinv2d, w1, b1)


def _tc_layer2(s2, v2, dinv2d, w2, b2, wr, br):
    def body(s2_ref, v2_ref, dinv_ref, w2_ref, b2_ref, wr_ref, br_ref, o_ref):
        dinv = dinv_ref[...]
        ah = jnp.concatenate(
            [s2_ref[0, 0:N, :] + v2_ref[0:N, :],
             s2_ref[1, 0:N, :] + v2_ref[N:2 * N, :]], axis=1) * dinv
        z = jnp.dot(ah, w2_ref[...], preferred_element_type=jnp.float32)
        h2 = jnp.maximum(z + b2_ref[...], 0.0)
        logit = jnp.dot(h2, wr_ref[...], preferred_element_type=jnp.float32)
        logit = logit + br_ref[...]
        o_ref[...] = 4.0 / (1.0 + jnp.exp(-logit))

    return pl.pallas_call(
        body,
        out_shape=jax.ShapeDtypeStruct((N, 1), jnp.float32),
    )(s2, v2, dinv2d, w2, b2, wr, br)


def kernel(x, edge_index, W1, b1, W2, b2, Wr, br):
    ei = edge_index.astype(jnp.int32)
    row, col = ei[0], ei[1]

    e1 = 32 * 5 * 16 * 128
    e1_sc0 = 16 * G1_SC0 * 16 * 128
    rowp = jnp.concatenate([row, jnp.zeros((e1 - E,), jnp.int32)])
    colp = jnp.concatenate([col, jnp.full((e1 - E,), JUNK, jnp.int32)])
    row1 = jnp.concatenate(
        [rowp[:e1_sc0], rowp[e1_sc0:] + N]).reshape(-1, 128)
    col1 = colp.reshape(-1, 128)

    row2 = jnp.concatenate([rowp, rowp + N]).reshape(-1, 128)
    col2 = jnp.concatenate([colp, colp]).reshape(-1, 128)

    cold = colp.reshape(32, DEG_CHUNKS, 128)

    zeros128 = jnp.zeros((ROWS_PER_SUB, 128), jnp.float32)
    ones128 = jnp.ones((128, 128), jnp.float32)

    d = _sc_degree(cold, zeros128, ones128)

    v1, dinv2d = _tc_prescale(d, x)
    s1 = _sc_aggregate(row1, col1, v1, zeros128, G1_SC0, G1_SC1)

    v2 = _tc_layer1(s1, v1, dinv2d, W1, b1.reshape(1, D_HID))
    s2 = _sc_aggregate(row2, col2, v2, zeros128, 10, 10)

    return _tc_layer2(s2, v2, dinv2d, W2, b2.reshape(1, D_HID),
                      Wr, br.reshape(1, 1))


# speedup vs baseline: 1.3368x; 1.2569x over previous
import functools

import jax
import jax.numpy as jnp
from jax import lax
from jax.experimental import pallas as pl
from jax.experimental.pallas import tpu as pltpu
from jax.experimental.pallas import tpu_sc as plsc

N = 10000
E = 320000
D_IN = 128
D_HID = 256
R_PAD = 10112
JUNK = N
NS = 16
ROWS_PER_SUB = R_PAD // NS

DEG_CHUNKS = 80
E_DEG = 32 * DEG_CHUNKS * 128

G1_SC0 = 5
G1_SC1 = 5

_mesh = lambda: plsc.VectorSubcoreMesh(core_axis_name="c", subcore_axis_name="s")


def _sc_degree(cold, zeros128, ones128):
    @functools.partial(
        pl.kernel,
        out_type=jax.ShapeDtypeStruct((2, R_PAD, 128), jnp.float32),
        mesh=_mesh(),
        scratch_types=[
            pltpu.VMEM((DEG_CHUNKS, 128), jnp.int32),
            pltpu.VMEM((128, 128), jnp.float32),
            pltpu.VMEM_SHARED((R_PAD, 128), jnp.float32),
            pltpu.SemaphoreType.DMA,
        ],
    )
    def deg_kernel(col_hbm, z_hbm, ones_hbm, out, cidx, ones_v, acc, dsem):
        cid = lax.axis_index("c")
        sid = lax.axis_index("s")
        w = cid * NS + sid
        pltpu.sync_copy(col_hbm.at[w], cidx)
        pltpu.sync_copy(ones_hbm, ones_v)
        sl = pl.ds(sid * ROWS_PER_SUB, ROWS_PER_SUB)
        pltpu.sync_copy(z_hbm, acc.at[sl])
        plsc.subcore_barrier()

        def wave(t, carry):
            for k in range(8):
                pltpu.async_copy(ones_v, acc.at[cidx.at[t * 8 + k]], dsem,
                                 add=True)
            for k in range(8):
                pltpu.make_async_copy(
                    ones_v, acc.at[cidx.at[t * 8 + k]], dsem).wait()
            return carry

        lax.fori_loop(0, DEG_CHUNKS // 8, wave, 0)
        plsc.subcore_barrier()
        pltpu.sync_copy(acc.at[sl], out.at[cid].at[sl])

    return deg_kernel(cold, zeros128, ones128)


_SPLIT = 4


def _issue_gather(t_hbm, ridx, j, gbuf, sem):
    step = 128 // _SPLIT
    for p in range(_SPLIT):
        pltpu.async_copy(
            t_hbm.at[ridx.at[j].at[pl.ds(p * step, step)]],
            gbuf.at[pl.ds(p * step, step)], sem)


def _wait_gather(t_hbm, ridx, j, gbuf, sem):
    step = 128 // _SPLIT
    for p in range(_SPLIT):
        pltpu.make_async_copy(
            t_hbm.at[ridx.at[j].at[pl.ds(p * step, step)]],
            gbuf.at[pl.ds(p * step, step)], sem).wait()


def _sc_aggregate(row2d, col2d, table, zeros128, g0, g1):
    @functools.partial(
        pl.kernel,
        out_type=jax.ShapeDtypeStruct((2, R_PAD, 128), jnp.float32),
        mesh=_mesh(),
        scratch_types=[
            pltpu.VMEM((16, 128), jnp.int32),
            pltpu.VMEM((16, 128), jnp.int32),
            pltpu.VMEM((128, 128), jnp.float32),
            pltpu.VMEM((128, 128), jnp.float32),
            pltpu.VMEM_SHARED((R_PAD, 128), jnp.float32),
            pltpu.SemaphoreType.DMA,
            pltpu.SemaphoreType.DMA,
            pltpu.SemaphoreType.DMA,
            pltpu.SemaphoreType.DMA,
        ],
    )
    def agg_kernel(row_hbm, col_hbm, t_hbm, z_hbm, out,
                   ridx, cidx, gbuf0, gbuf1, acc, gsem0, gsem1, ssem0, ssem1):
        cid = lax.axis_index("c")
        sid = lax.axis_index("s")
        base = (1 - cid) * (sid * g0 * 16) + cid * (NS * g0 * 16 +
                                                    sid * g1 * 16)
        ngroups = g0 + cid * (g1 - g0)
        sl = pl.ds(sid * ROWS_PER_SUB, ROWS_PER_SUB)
        pltpu.sync_copy(z_hbm, acc.at[sl])
        plsc.subcore_barrier()

        def group(g, carry):
            pltpu.sync_copy(row_hbm.at[pl.ds(base + g * 16, 16)], ridx)
            pltpu.sync_copy(col_hbm.at[pl.ds(base + g * 16, 16)], cidx)
            _issue_gather(t_hbm, ridx, 0, gbuf0, gsem0)
            _issue_gather(t_hbm, ridx, 1, gbuf1, gsem1)

            def steady(t, c2):
                j = 2 * t
                _wait_gather(t_hbm, ridx, j, gbuf0, gsem0)
                pltpu.async_copy(gbuf0, acc.at[cidx.at[j]], ssem0, add=True)
                pltpu.make_async_copy(gbuf0, acc.at[cidx.at[j]], ssem0).wait()
                _issue_gather(t_hbm, ridx, j + 2, gbuf0, gsem0)
                _wait_gather(t_hbm, ridx, j + 1, gbuf1, gsem1)
                pltpu.async_copy(gbuf1, acc.at[cidx.at[j + 1]], ssem1, add=True)
                pltpu.make_async_copy(
                    gbuf1, acc.at[cidx.at[j + 1]], ssem1).wait()
                _issue_gather(t_hbm, ridx, j + 3, gbuf1, gsem1)
                return c2

            lax.fori_loop(0, 7, steady, carry)
            _wait_gather(t_hbm, ridx, 14, gbuf0, gsem0)
            pltpu.sync_copy(gbuf0, acc.at[cidx.at[14]], add=True)
            _wait_gather(t_hbm, ridx, 15, gbuf1, gsem1)
            pltpu.sync_copy(gbuf1, acc.at[cidx.at[15]], add=True)
            return carry

        lax.fori_loop(0, ngroups, group, 0)
        plsc.subcore_barrier()
        pltpu.sync_copy(acc.at[sl], out.at[cid].at[sl])

    return agg_kernel(row2d, col2d, table, zeros128)


def _tc_prescale(d, x):
    def body(d_ref, x_ref, v_ref, dinv_ref):
        deg = d_ref[0, 0:N, 0:1] + d_ref[1, 0:N, 0:1] + 1.0
        dinv = lax.rsqrt(deg)
        dinv_ref[...] = dinv
        v = x_ref[...] * dinv
        v_ref[0:N, :] = v
        v_ref[N:2 * N, :] = v
        v_ref[2 * N:3 * N, :] = v
        v_ref[3 * N:4 * N, :] = v

    return pl.pallas_call(
        body,
        out_shape=(jax.ShapeDtypeStruct((4 * N, 128), jnp.float32),
                   jax.ShapeDtypeStruct((N, 1), jnp.float32)),
    )(d, x)


def _tc_layer1(s1, v1, dinv2d, w1, b1):
    def body(s_ref, v1_ref, dinv_ref, w1_ref, b1_ref, v2_ref):
        dinv = dinv_ref[...]
        ax = (s_ref[0, 0:N, :] + s_ref[1, 0:N, :] + v1_ref[0:N, :]) * dinv
        h = jnp.dot(ax, w1_ref[...], preferred_element_type=jnp.float32)
        h = jnp.maximum(h + b1_ref[...], 0.0) * dinv
        v2_ref[0:N, :] = h[:, 0:128]
        v2_ref[N:2 * N, :] = h[:, 128:256]
        v2_ref[2 * N:3 * N, :] = h[:, 0:128]
        v2_ref[3 * N:4 * N, :] = h[:, 128:256]

    return pl.pallas_call(
        body,
        out_shape=jax.ShapeDtypeStruct((4 * N, 128), jnp.float32),
    )(s1, v1, dinv2d, w1, b1)


def _tc_layer2(s2, v2, dinv2d, w2, b2, wr, br):
    def body(s2_ref, v2_ref, dinv_ref, w2_ref, b2_ref, wr_ref, br_ref, o_ref):
        dinv = dinv_ref[...]
        ah = jnp.concatenate(
            [s2_ref[0, 0:N, :] + v2_ref[0:N, :],
             s2_ref[1, 0:N, :] + v2_ref[N:2 * N, :]], axis=1) * dinv
        z = jnp.dot(ah, w2_ref[...], preferred_element_type=jnp.float32)
        h2 = jnp.maximum(z + b2_ref[...], 0.0)
        logit = jnp.dot(h2, wr_ref[...], preferred_element_type=jnp.float32)
        logit = logit + br_ref[...]
        o_ref[...] = 4.0 / (1.0 + jnp.exp(-logit))

    return pl.pallas_call(
        body,
        out_shape=jax.ShapeDtypeStruct((N, 1), jnp.float32),
    )(s2, v2, dinv2d, w2, b2, wr, br)


def kernel(x, edge_index, W1, b1, W2, b2, Wr, br):
    ei = edge_index.astype(jnp.int32)
    row, col = ei[0], ei[1]

    e1 = 32 * 5 * 16 * 128
    e1_sc0 = 16 * G1_SC0 * 16 * 128
    rowp = jnp.concatenate([row, jnp.zeros((e1 - E,), jnp.int32)])
    colp = jnp.concatenate([col, jnp.full((e1 - E,), JUNK, jnp.int32)])
    par1 = (jnp.arange(e1, dtype=jnp.int32) // 128) % 2
    off1 = jnp.where(jnp.arange(e1) < e1_sc0, par1 * (2 * N),
                     N + par1 * (2 * N)).astype(jnp.int32)
    row1 = (rowp + off1).reshape(-1, 128)
    col1 = colp.reshape(-1, 128)

    off2 = (par1 * (2 * N)).astype(jnp.int32)
    row2 = jnp.concatenate([rowp + off2, rowp + N + off2]).reshape(-1, 128)
    col2 = jnp.concatenate([colp, colp]).reshape(-1, 128)

    cold = colp.reshape(32, DEG_CHUNKS, 128)

    zeros128 = jnp.zeros((ROWS_PER_SUB, 128), jnp.float32)
    ones128 = jnp.ones((128, 128), jnp.float32)

    d = _sc_degree(cold, zeros128, ones128)

    v1, dinv2d = _tc_prescale(d, x)
    s1 = _sc_aggregate(row1, col1, v1, zeros128, G1_SC0, G1_SC1)

    v2 = _tc_layer1(s1, v1, dinv2d, W1, b1.reshape(1, D_HID))
    s2 = _sc_aggregate(row2, col2, v2, zeros128, 10, 10)

    return _tc_layer2(s2, v2, dinv2d, W2, b2.reshape(1, D_HID),
                      Wr, br.reshape(1, 1))


# speedup vs baseline: 1.3458x; 1.0067x over previous
import functools

import jax
import jax.numpy as jnp
from jax import lax
from jax.experimental import pallas as pl
from jax.experimental.pallas import tpu as pltpu
from jax.experimental.pallas import tpu_sc as plsc

N = 10000
E = 320000
D_IN = 128
D_HID = 256
R_PAD = 10112
JUNK = N
NS = 16
ROWS_PER_SUB = R_PAD // NS

DEG_CHUNKS = 80
E_DEG = 32 * DEG_CHUNKS * 128

G1_SC0 = 5
G1_SC1 = 5

_mesh = lambda: plsc.VectorSubcoreMesh(core_axis_name="c", subcore_axis_name="s")


def _sc_degree(cold, zeros128, ones128):
    @functools.partial(
        pl.kernel,
        out_type=jax.ShapeDtypeStruct((2, R_PAD, 128), jnp.float32),
        mesh=_mesh(),
        scratch_types=[
            pltpu.VMEM((DEG_CHUNKS, 128), jnp.int32),
            pltpu.VMEM((128, 128), jnp.float32),
            pltpu.VMEM_SHARED((R_PAD, 128), jnp.float32),
            pltpu.SemaphoreType.DMA,
        ],
    )
    def deg_kernel(col_hbm, z_hbm, ones_hbm, out, cidx, ones_v, acc, dsem):
        cid = lax.axis_index("c")
        sid = lax.axis_index("s")
        w = cid * NS + sid
        pltpu.sync_copy(col_hbm.at[w], cidx)
        pltpu.sync_copy(ones_hbm, ones_v)
        sl = pl.ds(sid * ROWS_PER_SUB, ROWS_PER_SUB)
        pltpu.sync_copy(z_hbm, acc.at[sl])
        plsc.subcore_barrier()

        def wave(t, carry):
            for k in range(8):
                pltpu.async_copy(ones_v, acc.at[cidx.at[t * 8 + k]], dsem,
                                 add=True)
            for k in range(8):
                pltpu.make_async_copy(
                    ones_v, acc.at[cidx.at[t * 8 + k]], dsem).wait()
            return carry

        lax.fori_loop(0, DEG_CHUNKS // 8, wave, 0)
        plsc.subcore_barrier()
        pltpu.sync_copy(acc.at[sl], out.at[cid].at[sl])

    return deg_kernel(cold, zeros128, ones128)


_SPLIT = 4


def _issue_gather(t_hbm, ridx, j, gbuf, sem):
    step = 128 // _SPLIT
    for p in range(_SPLIT):
        pltpu.async_copy(
            t_hbm.at[ridx.at[j].at[pl.ds(p * step, step)]],
            gbuf.at[pl.ds(p * step, step)], sem)


def _wait_gather(t_hbm, ridx, j, gbuf, sem):
    step = 128 // _SPLIT
    for p in range(_SPLIT):
        pltpu.make_async_copy(
            t_hbm.at[ridx.at[j].at[pl.ds(p * step, step)]],
            gbuf.at[pl.ds(p * step, step)], sem).wait()


def _sc_aggregate(row2d, col2d, table, zeros128, g0, g1):
    @functools.partial(
        pl.kernel,
        out_type=jax.ShapeDtypeStruct((2, R_PAD, 128), jnp.float32),
        mesh=_mesh(),
        scratch_types=[
            pltpu.VMEM((16, 128), jnp.int32),
            pltpu.VMEM((16, 128), jnp.int32),
            pltpu.VMEM((128, 128), jnp.float32),
            pltpu.VMEM((128, 128), jnp.float32),
            pltpu.VMEM_SHARED((R_PAD, 128), jnp.float32),
            pltpu.SemaphoreType.DMA,
            pltpu.SemaphoreType.DMA,
            pltpu.SemaphoreType.DMA,
            pltpu.SemaphoreType.DMA,
        ],
    )
    def agg_kernel(row_hbm, col_hbm, t_hbm, z_hbm, out,
                   ridx, cidx, gbuf0, gbuf1, acc, gsem0, gsem1, ssem0, ssem1):
        cid = lax.axis_index("c")
        sid = lax.axis_index("s")
        base = (1 - cid) * (sid * g0 * 16) + cid * (NS * g0 * 16 +
                                                    sid * g1 * 16)
        ngroups = g0 + cid * (g1 - g0)
        sl = pl.ds(sid * ROWS_PER_SUB, ROWS_PER_SUB)
        pltpu.sync_copy(z_hbm, acc.at[sl])
        plsc.subcore_barrier()

        def group(g, carry):
            pltpu.sync_copy(row_hbm.at[pl.ds(base + g * 16, 16)], ridx)
            pltpu.sync_copy(col_hbm.at[pl.ds(base + g * 16, 16)], cidx)
            _issue_gather(t_hbm, ridx, 0, gbuf0, gsem0)
            _issue_gather(t_hbm, ridx, 1, gbuf1, gsem1)

            def steady(t, c2):
                j = 2 * t
                _wait_gather(t_hbm, ridx, j, gbuf0, gsem0)
                pltpu.async_copy(gbuf0, acc.at[cidx.at[j]], ssem0, add=True)
                pltpu.make_async_copy(gbuf0, acc.at[cidx.at[j]], ssem0).wait()
                _issue_gather(t_hbm, ridx, j + 2, gbuf0, gsem0)
                _wait_gather(t_hbm, ridx, j + 1, gbuf1, gsem1)
                pltpu.async_copy(gbuf1, acc.at[cidx.at[j + 1]], ssem1, add=True)
                pltpu.make_async_copy(
                    gbuf1, acc.at[cidx.at[j + 1]], ssem1).wait()
                _issue_gather(t_hbm, ridx, j + 3, gbuf1, gsem1)
                return c2

            lax.fori_loop(0, 7, steady, carry)
            _wait_gather(t_hbm, ridx, 14, gbuf0, gsem0)
            pltpu.sync_copy(gbuf0, acc.at[cidx.at[14]], add=True)
            _wait_gather(t_hbm, ridx, 15, gbuf1, gsem1)
            pltpu.sync_copy(gbuf1, acc.at[cidx.at[15]], add=True)
            return carry

        lax.fori_loop(0, ngroups, group, 0)
        plsc.subcore_barrier()
        pltpu.sync_copy(acc.at[sl], out.at[cid].at[sl])

    return agg_kernel(row2d, col2d, table, zeros128)


def _tc_prescale(d, x):
    def body(d_ref, x_ref, v_ref, dinv_ref):
        deg = d_ref[0, 0:N, 0:1] + d_ref[1, 0:N, 0:1] + 1.0
        dinv = lax.rsqrt(deg)
        dinv_ref[...] = dinv
        v_ref[...] = x_ref[...] * dinv

    return pl.pallas_call(
        body,
        grid=(8,),
        in_specs=[pl.BlockSpec((2, R_PAD, 128), lambda k: (0, 0, 0)),
                  pl.BlockSpec((N, 128), lambda k: (0, 0))],
        out_specs=(pl.BlockSpec((N, 128), lambda k: (k, 0)),
                   pl.BlockSpec((N, 1), lambda k: (0, 0))),
        out_shape=(jax.ShapeDtypeStruct((8 * N, 128), jnp.float32),
                   jax.ShapeDtypeStruct((N, 1), jnp.float32)),
    )(d, x)


def _tc_layer1(s1, v1, dinv2d, w1, b1):
    def body(s_ref, v1_ref, dinv_ref, w1_ref, b1_ref, v2_ref):
        k = pl.program_id(0)
        dinv = dinv_ref[...]
        ax = (s_ref[0, 0:N, :] + s_ref[1, 0:N, :] + v1_ref[...]) * dinv
        h = jnp.dot(ax, w1_ref[...], preferred_element_type=jnp.float32)
        h = jnp.maximum(h + b1_ref[...], 0.0) * dinv
        v2_ref[...] = jnp.where((k % 2) == 0, h[:, 0:128], h[:, 128:256])

    return pl.pallas_call(
        body,
        grid=(8,),
        in_specs=[pl.BlockSpec((2, R_PAD, 128), lambda k: (0, 0, 0)),
                  pl.BlockSpec((N, 128), lambda k: (0, 0)),
                  pl.BlockSpec((N, 1), lambda k: (0, 0)),
                  pl.BlockSpec((D_IN, D_HID), lambda k: (0, 0)),
                  pl.BlockSpec((1, D_HID), lambda k: (0, 0))],
        out_specs=pl.BlockSpec((N, 128), lambda k: (k, 0)),
        out_shape=jax.ShapeDtypeStruct((8 * N, 128), jnp.float32),
    )(s1, v1, dinv2d, w1, b1)


def _tc_layer2(s2, v2, dinv2d, w2, b2, wr, br):
    def body(s2_ref, v2_ref, dinv_ref, w2_ref, b2_ref, wr_ref, br_ref, o_ref):
        dinv = dinv_ref[...]
        ah = jnp.concatenate(
            [s2_ref[0, 0:N, :] + v2_ref[0:N, :],
             s2_ref[1, 0:N, :] + v2_ref[N:2 * N, :]], axis=1) * dinv
        z = jnp.dot(ah, w2_ref[...], preferred_element_type=jnp.float32)
        h2 = jnp.maximum(z + b2_ref[...], 0.0)
        logit = jnp.dot(h2, wr_ref[...], preferred_element_type=jnp.float32)
        logit = logit + br_ref[...]
        o_ref[...] = 4.0 / (1.0 + jnp.exp(-logit))

    return pl.pallas_call(
        body,
        grid=(1,),
        in_specs=[pl.BlockSpec((2, R_PAD, 128), lambda k: (0, 0, 0)),
                  pl.BlockSpec((2 * N, 128), lambda k: (0, 0)),
                  pl.BlockSpec((N, 1), lambda k: (0, 0)),
                  pl.BlockSpec((D_HID, D_HID), lambda k: (0, 0)),
                  pl.BlockSpec((1, D_HID), lambda k: (0, 0)),
                  pl.BlockSpec((D_HID, 1), lambda k: (0, 0)),
                  pl.BlockSpec((1, 1), lambda k: (0, 0))],
        out_specs=pl.BlockSpec((N, 1), lambda k: (0, 0)),
        out_shape=jax.ShapeDtypeStruct((N, 1), jnp.float32),
    )(s2, v2, dinv2d, w2, b2, wr, br)


def kernel(x, edge_index, W1, b1, W2, b2, Wr, br):
    ei = edge_index.astype(jnp.int32)
    row, col = ei[0], ei[1]

    e1 = 32 * 5 * 16 * 128
    e1_sc0 = 16 * G1_SC0 * 16 * 128
    rowp = jnp.concatenate([row, jnp.zeros((e1 - E,), jnp.int32)])
    colp = jnp.concatenate([col, jnp.full((e1 - E,), JUNK, jnp.int32)])
    par1 = (jnp.arange(e1, dtype=jnp.int32) // 128) % 4
    off1 = jnp.where(jnp.arange(e1) < e1_sc0, par1 * (2 * N),
                     N + par1 * (2 * N)).astype(jnp.int32)
    row1 = (rowp + off1).reshape(-1, 128)
    col1 = colp.reshape(-1, 128)

    off2 = (par1 * (2 * N)).astype(jnp.int32)
    row2 = jnp.concatenate([rowp + off2, rowp + N + off2]).reshape(-1, 128)
    col2 = jnp.concatenate([colp, colp]).reshape(-1, 128)

    cold = colp.reshape(32, DEG_CHUNKS, 128)

    zeros128 = jnp.zeros((ROWS_PER_SUB, 128), jnp.float32)
    ones128 = jnp.ones((128, 128), jnp.float32)

    d = _sc_degree(cold, zeros128, ones128)

    v1, dinv2d = _tc_prescale(d, x)
    s1 = _sc_aggregate(row1, col1, v1, zeros128, G1_SC0, G1_SC1)

    v2 = _tc_layer1(s1, v1, dinv2d, W1, b1.reshape(1, D_HID))
    s2 = _sc_aggregate(row2, col2, v2, zeros128, 10, 10)

    return _tc_layer2(s2, v2, dinv2d, W2, b2.reshape(1, D_HID),
                      Wr, br.reshape(1, 1))
